# baseline (speedup 1.0000x reference)
import jax
import jax.numpy as jnp
from jax.experimental import pallas as pl
from jax.experimental.pallas import tpu as pltpu

_THRESHOLD = 0.5
_BLOCK_ROWS = 512


def _threshold_kernel(f_ref, o_ref):
    f = f_ref[...]
    o_ref[...] = jnp.where(f > _THRESHOLD, jnp.zeros_like(f), f)


def kernel(features, W1, b1, W2, b2):
    del W1, b1, W2, b2
    B, D = features.shape
    block_cols = 2048
    return pl.pallas_call(
        _threshold_kernel,
        grid=(B // 1024, D // block_cols),
        in_specs=[pl.BlockSpec((1024, block_cols), lambda i, j: (i, j))],
        out_specs=pl.BlockSpec((1024, block_cols), lambda i, j: (i, j)),
        out_shape=jax.ShapeDtypeStruct((B, D), features.dtype),
        compiler_params=pltpu.CompilerParams(
            dimension_semantics=("parallel", "parallel"),
        ),
    )(features)


# speedup vs baseline: 1.0031x; 1.0031x over previous
import jax
import jax.numpy as jnp
from jax.experimental import pallas as pl
from jax.experimental.pallas import tpu as pltpu

_THRESHOLD = 0.5
_BLOCK_ROWS = 512


def _threshold_kernel(f_ref, o_ref):
    f = f_ref[...]
    o_ref[...] = jnp.where(f > _THRESHOLD, jnp.zeros_like(f), f)


def kernel(features, W1, b1, W2, b2):
    del W1, b1, W2, b2
    B, D = features.shape
    return pl.pallas_call(
        _threshold_kernel,
        grid=(B // _BLOCK_ROWS,),
        in_specs=[pl.BlockSpec((_BLOCK_ROWS, D), lambda i: (i, 0))],
        out_specs=pl.BlockSpec((_BLOCK_ROWS, D), lambda i: (i, 0)),
        out_shape=jax.ShapeDtypeStruct((B, D), features.dtype),
        compiler_params=pltpu.CompilerParams(
            dimension_semantics=("parallel",),
        ),
    )(features)
